# Initial kernel scaffold; baseline (speedup 1.0000x reference)
import functools

import jax
import jax.numpy as jnp
from jax import lax
from jax.experimental import pallas as pl
from jax.experimental.pallas import tpu as pltpu
from jax.experimental.pallas import tpu_sc as plsc

N = 10000
E = 320000
D = 128

NC = 2
NS = 16
NW = NC * NS
EPT = E // NW
K = 80
NB = EPT // K
RPT = N // NS
ZR = 125


_BN = 1000


def _matmul_body(x_ref, w_ref, o_ref):
    o_ref[...] = jnp.dot(x_ref[...], w_ref[...],
                         preferred_element_type=jnp.float32)


def _matmul(x, W):
    return pl.pallas_call(
        _matmul_body,
        grid=(N // _BN,),
        in_specs=[
            pl.BlockSpec((_BN, D), lambda i: (i, 0)),
            pl.BlockSpec((D, D), lambda i: (0, 0)),
        ],
        out_specs=pl.BlockSpec((_BN, D), lambda i: (i, 0)),
        out_shape=jax.ShapeDtypeStruct((N, D), jnp.float32),
    )(x, W)


def _sc_body(support_hbm, src_hbm, dst_hbm, w_hbm, out_hbm,
             acc, idx_v, dst_v, w_v, rows_v, zbuf, gsem):
    c = lax.axis_index("c")
    s = lax.axis_index("s")
    wid = c * NS + s

    zv = jnp.zeros((16,), jnp.float32)

    def zero_row(i, _):
        for cc in range(D // 16):
            zbuf[i, pl.ds(cc * 16, 16)] = zv
        return 0

    lax.fori_loop(0, ZR, zero_row, 0)
    for t in range(RPT // ZR):
        pltpu.sync_copy(zbuf, acc.at[pl.ds(s * RPT + t * ZR, ZR)])
    plsc.subcore_barrier()

    ebase = wid * EPT

    def block(bi, _):
        off = ebase + bi * K
        pltpu.sync_copy(src_hbm.at[pl.ds(off, K)], idx_v)
        pltpu.sync_copy(dst_hbm.at[pl.ds(off, K)], dst_v)
        pltpu.sync_copy(w_hbm.at[pl.ds(off, K)], w_v)
        pltpu.async_copy(support_hbm.at[idx_v], rows_v, gsem).wait()

        def edge(j, _):
            wj = w_v[j]
            for cc in range(D // 16):
                sl = pl.ds(cc * 16, 16)
                rows_v[j, sl] = rows_v[j, sl] * wj
            return 0

        lax.fori_loop(0, K, edge, 0)
        pltpu.sync_copy(rows_v, acc.at[dst_v], add=True)
        return 0

    lax.fori_loop(0, NB, block, 0)
    plsc.subcore_barrier()

    pltpu.sync_copy(acc.at[pl.ds(s * RPT, RPT)],
                    out_hbm.at[c, pl.ds(s * RPT, RPT)])


def _sc_aggregate(support, src, dst, w):
    mesh = plsc.VectorSubcoreMesh(core_axis_name="c", subcore_axis_name="s",
                                  num_cores=NC, num_subcores=NS)
    return pl.kernel(
        _sc_body,
        out_type=jax.ShapeDtypeStruct((NC, N, D), jnp.float32),
        mesh=mesh,
        scratch_types=[
            pltpu.VMEM_SHARED((N, D), jnp.float32),
            pltpu.VMEM((K,), jnp.int32),
            pltpu.VMEM((K,), jnp.int32),
            pltpu.VMEM((K,), jnp.float32),
            pltpu.VMEM((K, D), jnp.float32),
            pltpu.VMEM((ZR, D), jnp.float32),
            pltpu.SemaphoreType.DMA,
        ],
    )(support, src, dst, w)


def _combine_body(p_ref, b_ref, o_ref):
    o_ref[...] = p_ref[0] + p_ref[1] + b_ref[...]


def _combine(partials, b):
    return pl.pallas_call(
        _combine_body,
        grid=(N // _BN,),
        in_specs=[
            pl.BlockSpec((NC, _BN, D), lambda i: (0, i, 0)),
            pl.BlockSpec((1, D), lambda i: (0, 0)),
        ],
        out_specs=pl.BlockSpec((_BN, D), lambda i: (i, 0)),
        out_shape=jax.ShapeDtypeStruct((N, D), jnp.float32),
    )(partials, b.reshape(1, D))


def kernel(input, edge_index, edge_weight, W, b):
    support = _matmul(input, W)
    partials = _sc_aggregate(support, edge_index[0], edge_index[1],
                             edge_weight)
    return _combine(partials, b)


# speedup vs baseline: 4.4493x; 4.4493x over previous
import functools

import jax
import jax.numpy as jnp
from jax import lax
from jax.experimental import pallas as pl
from jax.experimental.pallas import tpu as pltpu
from jax.experimental.pallas import tpu_sc as plsc

N = 10000
E = 320000
D = 128

NC = 2
NS = 16
NW = NC * NS
EPT = E // NW
K = 80
NB = EPT // K
RPT = 624
REM = N - NS * RPT
ZR = 16


_BN = 1000


def _matmul_body(x_ref, w_ref, o_ref):
    o_ref[...] = jnp.dot(x_ref[...], w_ref[...],
                         preferred_element_type=jnp.float32)


def _matmul(x, W):
    return pl.pallas_call(
        _matmul_body,
        grid=(N // _BN,),
        in_specs=[
            pl.BlockSpec((_BN, D), lambda i: (i, 0)),
            pl.BlockSpec((D, D), lambda i: (0, 0)),
        ],
        out_specs=pl.BlockSpec((_BN, D), lambda i: (i, 0)),
        out_shape=jax.ShapeDtypeStruct((N, D), jnp.float32),
    )(x, W)


def _sc_body(support_hbm, src_hbm, dst_hbm, w_hbm, out_hbm,
             acc, idx_v, dst_v, w_v, rows_v, zbuf, gsem):
    c = lax.axis_index("c")
    s = lax.axis_index("s")
    wid = c * NS + s

    zv = jnp.zeros((16,), jnp.float32)

    def zero_row(i, _):
        for cc in range(D // 16):
            zbuf[i, pl.ds(cc * 16, 16)] = zv
        return 0

    lax.fori_loop(0, ZR, zero_row, 0)
    for t in range(RPT // ZR):
        pltpu.sync_copy(zbuf, acc.at[pl.ds(s * RPT + t * ZR, ZR)])

    @pl.when(s == 0)
    def _():
        pltpu.sync_copy(zbuf, acc.at[pl.ds(NS * RPT, REM)])

    plsc.subcore_barrier()

    ebase = wid * EPT

    def block(bi, _):
        off = ebase + bi * K
        pltpu.sync_copy(src_hbm.at[pl.ds(off, K)], idx_v)
        pltpu.sync_copy(dst_hbm.at[pl.ds(off, K)], dst_v)
        pltpu.sync_copy(w_hbm.at[pl.ds(off, K)], w_v)
        pltpu.async_copy(support_hbm.at[idx_v], rows_v, gsem).wait()

        def egroup(g, _):
            wv = w_v[pl.ds(g * 16, 16)]
            for j16 in range(16):
                wj = wv[j16]
                j = g * 16 + j16
                for cc in range(D // 16):
                    sl = pl.ds(cc * 16, 16)
                    rows_v[j, sl] = rows_v[j, sl] * wj
            return 0

        lax.fori_loop(0, K // 16, egroup, 0)
        pltpu.sync_copy(rows_v, acc.at[dst_v], add=True)
        return 0

    lax.fori_loop(0, NB, block, 0)
    plsc.subcore_barrier()

    pltpu.sync_copy(acc.at[pl.ds(s * RPT, RPT)],
                    out_hbm.at[c, pl.ds(s * RPT, RPT)])

    @pl.when(s == 0)
    def _():
        pltpu.sync_copy(acc.at[pl.ds(NS * RPT, REM)],
                        out_hbm.at[c, pl.ds(NS * RPT, REM)])


def _sc_aggregate(support, src, dst, w):
    mesh = plsc.VectorSubcoreMesh(core_axis_name="c", subcore_axis_name="s",
                                  num_cores=NC, num_subcores=NS)
    return pl.kernel(
        _sc_body,
        out_type=jax.ShapeDtypeStruct((NC, N, D), jnp.float32),
        mesh=mesh,
        scratch_types=[
            pltpu.VMEM_SHARED((N, D), jnp.float32),
            pltpu.VMEM((K,), jnp.int32),
            pltpu.VMEM((K,), jnp.int32),
            pltpu.VMEM((K,), jnp.float32),
            pltpu.VMEM((K, D), jnp.float32),
            pltpu.VMEM((ZR, D), jnp.float32),
            pltpu.SemaphoreType.DMA,
        ],
    )(support, src, dst, w)


def _combine_body(p_ref, b_ref, o_ref):
    o_ref[...] = p_ref[0] + p_ref[1] + b_ref[...]


def _combine(partials, b):
    return pl.pallas_call(
        _combine_body,
        grid=(N // _BN,),
        in_specs=[
            pl.BlockSpec((NC, _BN, D), lambda i: (0, i, 0)),
            pl.BlockSpec((1, D), lambda i: (0, 0)),
        ],
        out_specs=pl.BlockSpec((_BN, D), lambda i: (i, 0)),
        out_shape=jax.ShapeDtypeStruct((N, D), jnp.float32),
    )(partials, b.reshape(1, D))


def kernel(input, edge_index, edge_weight, W, b):
    support = _matmul(input, W)
    partials = _sc_aggregate(support, edge_index[0], edge_index[1],
                             edge_weight)
    return _combine(partials, b)


# speedup vs baseline: 7.4090x; 1.6652x over previous
import functools

import jax
import jax.numpy as jnp
from jax import lax
from jax.experimental import pallas as pl
from jax.experimental.pallas import tpu as pltpu
from jax.experimental.pallas import tpu_sc as plsc

N = 10000
E = 320000
D = 128

NC = 2
NS = 16
NW = NC * NS
EPT = E // NW
K = 40
NB = EPT // K
RPT = 624
REM = N - NS * RPT
ZR = 16


_BN = 1000


def _matmul_body(x_ref, w_ref, o_ref):
    o_ref[...] = jnp.dot(x_ref[...], w_ref[...],
                         preferred_element_type=jnp.float32)


def _matmul(x, W):
    return pl.pallas_call(
        _matmul_body,
        grid=(N // _BN,),
        in_specs=[
            pl.BlockSpec((_BN, D), lambda i: (i, 0)),
            pl.BlockSpec((D, D), lambda i: (0, 0)),
        ],
        out_specs=pl.BlockSpec((_BN, D), lambda i: (i, 0)),
        out_shape=jax.ShapeDtypeStruct((N, D), jnp.float32),
    )(x, W)


def _scale_rows(rows_v, w_all, bi):
    for g in range((K + 15) // 16):
        p = min(16, K - g * 16)
        lane0 = 16 - p
        wv = w_all[pl.ds(bi * K + g * 16 - lane0, 16)]
        for t in range(p):
            wj = wv[lane0 + t]
            j = g * 16 + t
            for cc in range(D // 16):
                sl = pl.ds(cc * 16, 16)
                rows_v[j, sl] = rows_v[j, sl] * wj


def _sc_body(support_hbm, src_hbm, dst_hbm, w_hbm, zeros_hbm, out_hbm,
             acc, src_all, dst_all, w_all, rows0, rows1,
             psem, g0, g1):
    c = lax.axis_index("c")
    s = lax.axis_index("s")
    wid = c * NS + s

    d1 = pltpu.async_copy(src_hbm.at[wid], src_all, psem)
    d2 = pltpu.async_copy(dst_hbm.at[wid], dst_all, psem)
    d3 = pltpu.async_copy(w_hbm.at[wid], w_all, psem)

    for t in range(RPT // ZR):
        pltpu.sync_copy(zeros_hbm, acc.at[pl.ds(s * RPT + t * ZR, ZR)])

    @pl.when(s == 0)
    def _():
        pltpu.sync_copy(zeros_hbm.at[pl.ds(0, REM)],
                        acc.at[pl.ds(NS * RPT, REM)])

    d1.wait()
    d2.wait()
    d3.wait()
    plsc.subcore_barrier()

    def gather(bi, buf, sem):
        return pltpu.async_copy(support_hbm.at[src_all.at[bi]], buf, sem)

    gather(0, rows0, g0)

    def pair(i, _):
        b0 = 2 * i

        gather(b0 + 1, rows1, g1)
        pltpu.make_async_copy(support_hbm.at[src_all.at[b0]], rows0, g0).wait()
        _scale_rows(rows0, w_all, b0)
        pltpu.sync_copy(rows0, acc.at[dst_all.at[b0]], add=True)

        @pl.when(b0 + 2 < NB)
        def _():
            gather(b0 + 2, rows0, g0)

        pltpu.make_async_copy(support_hbm.at[src_all.at[b0]], rows1, g1).wait()
        _scale_rows(rows1, w_all, b0 + 1)
        pltpu.sync_copy(rows1, acc.at[dst_all.at[b0 + 1]], add=True)
        return 0

    lax.fori_loop(0, NB // 2, pair, 0)
    if NB % 2:
        bl = NB - 1
        pltpu.make_async_copy(support_hbm.at[src_all.at[bl]], rows0,
                              g0).wait()
        _scale_rows(rows0, w_all, bl)
        pltpu.sync_copy(rows0, acc.at[dst_all.at[bl]], add=True)

    plsc.subcore_barrier()

    pltpu.sync_copy(acc.at[pl.ds(s * RPT, RPT)],
                    out_hbm.at[c, pl.ds(s * RPT, RPT)])

    @pl.when(s == 0)
    def _():
        pltpu.sync_copy(acc.at[pl.ds(NS * RPT, REM)],
                        out_hbm.at[c, pl.ds(NS * RPT, REM)])


def _sc_aggregate(support, src, dst, w):
    mesh = plsc.VectorSubcoreMesh(core_axis_name="c", subcore_axis_name="s",
                                  num_cores=NC, num_subcores=NS)
    return pl.kernel(
        _sc_body,
        out_type=jax.ShapeDtypeStruct((NC, N, D), jnp.float32),
        mesh=mesh,
        compiler_params=pltpu.CompilerParams(use_tc_tiling_on_sc=False),
        scratch_types=[
            pltpu.VMEM_SHARED((N, D), jnp.float32),
            pltpu.VMEM((NB, K), jnp.int32),
            pltpu.VMEM((NB, K), jnp.int32),
            pltpu.VMEM((EPT,), jnp.float32),
            pltpu.VMEM((K, D), jnp.float32),
            pltpu.VMEM((K, D), jnp.float32),
            pltpu.SemaphoreType.DMA,
            pltpu.SemaphoreType.DMA,
            pltpu.SemaphoreType.DMA,
        ],
    )(support.reshape(N, D),
      src.reshape(NW, NB, K), dst.reshape(NW, NB, K), w.reshape(NW, EPT),
      jnp.zeros((ZR, D), jnp.float32))


def _combine_body(p_ref, b_ref, o_ref):
    o_ref[...] = p_ref[0] + p_ref[1] + b_ref[...]


def _combine(partials, b):
    return pl.pallas_call(
        _combine_body,
        grid=(N // _BN,),
        in_specs=[
            pl.BlockSpec((NC, _BN, D), lambda i: (0, i, 0)),
            pl.BlockSpec((1, D), lambda i: (0, 0)),
        ],
        out_specs=pl.BlockSpec((_BN, D), lambda i: (i, 0)),
        out_shape=jax.ShapeDtypeStruct((N, D), jnp.float32),
    )(partials, b.reshape(1, D))


def kernel(input, edge_index, edge_weight, W, b):
    support = _matmul(input, W)
    partials = _sc_aggregate(support, edge_index[0], edge_index[1],
                             edge_weight)
    return _combine(partials, b)
